# baseline (speedup 1.0000x reference)
import functools

import jax
import jax.numpy as jnp
from jax import lax
from jax.experimental import pallas as pl
from jax.experimental.pallas import tpu as pltpu
from jax.experimental.pallas import tpu_sc as plsc

T = 32768
D = 768
OUT = 768
NC = 2
NS = 16
NW = NC * NS
CHUNK = T // NW
CT = 128
S = NW * CT
BM = 256
SP = BM + S
L = 16
UC = 4
UW = 64
CTU = UC * UW
ZN = CHUNK // UW

_mesh = plsc.VectorSubcoreMesh(core_axis_name="c", subcore_axis_name="s")
_sc_params = pltpu.CompilerParams(needs_layout_passes=False)


def _wid():
    return lax.axis_index("s") * NC + lax.axis_index("c")


def _iota():
    return lax.iota(jnp.int32, L)


@functools.partial(
    pl.kernel,
    mesh=_mesh,
    compiler_params=_sc_params,
    out_type=[
        jax.ShapeDtypeStruct((S,), jnp.int32),
        jax.ShapeDtypeStruct((S,), jnp.int32),
        jax.ShapeDtypeStruct((S,), jnp.int32),
        jax.ShapeDtypeStruct((NW, UC, UW), jnp.int32),
        jax.ShapeDtypeStruct((NW, UC, UW), jnp.int32),
        jax.ShapeDtypeStruct((NW, UC, UW), jnp.int32),
        jax.ShapeDtypeStruct((NW, UC, UW), jnp.float32),
        jax.ShapeDtypeStruct((NW * L,), jnp.int32),
    ],
    scratch_types=[
        pltpu.VMEM((CHUNK,), jnp.int32),
        pltpu.VMEM((CHUNK,), jnp.int32),
        pltpu.VMEM((CHUNK,), jnp.float32),
        pltpu.VMEM((CT,), jnp.int32),
        pltpu.VMEM((CT,), jnp.int32),
        pltpu.VMEM((CT,), jnp.int32),
        pltpu.VMEM((UC, UW), jnp.int32),
        pltpu.VMEM((UC, UW), jnp.int32),
        pltpu.VMEM((UC, UW), jnp.int32),
        pltpu.VMEM((UC, UW), jnp.float32),
        pltpu.VMEM((L,), jnp.int32),
    ],
)
def _k1_router(t0_hbm, t1_hbm, w_hbm,
               idx0_hbm, idx1_hbm, fp_hbm, idxU_hbm, pU0_hbm, pU1_hbm,
               wU_hbm, cnt_hbm,
               t0v, t1v, wcv, idx0l, idx1l, fpl, idxUl, pU0l, pU1l, wUl,
               infov):
    wid = _wid()
    base = wid * CHUNK
    iota = _iota()
    zi = jnp.zeros((L,), jnp.int32)
    zf = jnp.zeros((L,), jnp.float32)

    pltpu.sync_copy(t0_hbm.at[pl.ds(base, CHUNK)], t0v)
    pltpu.sync_copy(t1_hbm.at[pl.ds(base, CHUNK)], t1v)
    pltpu.sync_copy(w_hbm.at[pl.ds(base, CHUNK)], wcv)

    def init(j, _):
        idx0l[pl.ds(j * L, L)] = zi
        idx1l[pl.ds(j * L, L)] = zi
        fpl[pl.ds(j * L, L)] = zi
        return 0

    lax.fori_loop(0, CT // L, init, 0)

    for cc in range(UC):
        def initu(j, _, cc=cc):
            idxUl[cc, pl.ds(j * L, L)] = base + zi
            pU0l[cc, pl.ds(j * L, L)] = zi
            pU1l[cc, pl.ds(j * L, L)] = zi
            wUl[cc, pl.ds(j * L, L)] = zf
            return 0

        lax.fori_loop(0, UW // L, initu, 0)

    seg_base = BM + wid * CT

    def body(j, carry):
        s0, s1, sU = carry
        off = j * L
        v0 = t0v[pl.ds(off, L)]
        v1 = t1v[pl.ds(off, L)]
        wload = wcv[pl.ds(off, L)]
        m0 = (v0 == 0) | (v1 == 0)
        m1 = (v0 == 1) | (v1 == 1)
        mU = m0 | m1
        c0 = jnp.cumsum(jnp.where(m0, 1, 0)) + s0
        c1 = jnp.cumsum(jnp.where(m1, 1, 0)) + s1
        cU = jnp.cumsum(jnp.where(mU, 1, 0)) + sU
        r0 = c0 - 1
        r1 = c1 - 1
        rU = cU - 1
        tok = base + off + iota
        ok0 = m0 & (r0 < CT)
        ok1 = m1 & (r1 < CT)
        okU = mU & (rU < CTU)
        plsc.store_scatter(idx0l, [r0], tok, mask=ok0)
        plsc.store_scatter(idx1l, [r1], tok, mask=ok1)
        fpv = jnp.where(c0 > 0, BM + wid * CT + jnp.minimum(c0, CT) - 1,
                        -jnp.ones((L,), jnp.int32))
        plsc.store_scatter(fpl, [r1], fpv, mask=ok1)
        hi = lax.shift_right_logical(jnp.maximum(rU, 0), 6)
        lo = jnp.maximum(rU, 0) & 63
        plsc.store_scatter(idxUl, [hi, lo], tok, mask=okU)
        plsc.store_scatter(pU0l, [hi, lo], jnp.where(ok0, seg_base + r0, 0),
                           mask=okU)
        plsc.store_scatter(pU1l, [hi, lo], jnp.where(ok1, seg_base + r1, 0),
                           mask=okU)
        plsc.store_scatter(wUl, [hi, lo], wload, mask=okU)
        s0n = s0 + plsc.all_reduce_population_count(m0)
        s1n = s1 + plsc.all_reduce_population_count(m1)
        sUn = sU + plsc.all_reduce_population_count(mU)
        return s0n, s1n, sUn

    s0, s1, sU = lax.fori_loop(0, CHUNK // L, body, (zi, zi, zi))

    ltv = jnp.maximum(jnp.minimum(sU, CTU) - 1, 0)
    lhi = lax.shift_right_logical(ltv, 6)
    llo = ltv & 63
    lastTok = plsc.load_gather(idxUl, [lhi, llo])
    lastP0 = plsc.load_gather(pU0l, [lhi, llo])
    lastP1 = plsc.load_gather(pU1l, [lhi, llo])
    lastW = plsc.load_gather(wUl, [lhi, llo])
    for cc in range(UC):
        for jj in range(UW // L):
            slot = cc * UW + jj * L + iota
            cond = slot < sU
            sl = pl.ds(jj * L, L)
            idxUl[cc, sl] = jnp.where(cond, idxUl[cc, sl], lastTok)
            pU0l[cc, sl] = jnp.where(cond, pU0l[cc, sl], lastP0)
            pU1l[cc, sl] = jnp.where(cond, pU1l[cc, sl], lastP1)
            wUl[cc, sl] = jnp.where(cond, wUl[cc, sl], lastW)

    infov[...] = jnp.where(
        iota == 0, s0,
        jnp.where(iota == 1, s1, jnp.where(iota == 2, sU, 0)))
    pltpu.sync_copy(idx0l, idx0_hbm.at[pl.ds(wid * CT, CT)])
    pltpu.sync_copy(idx1l, idx1_hbm.at[pl.ds(wid * CT, CT)])
    pltpu.sync_copy(fpl, fp_hbm.at[pl.ds(wid * CT, CT)])
    pltpu.sync_copy(idxUl, idxU_hbm.at[wid])
    pltpu.sync_copy(pU0l, pU0_hbm.at[wid])
    pltpu.sync_copy(pU1l, pU1_hbm.at[wid])
    pltpu.sync_copy(wUl, wU_hbm.at[wid])
    pltpu.sync_copy(infov, cnt_hbm.at[pl.ds(wid * L, L)])


@functools.partial(
    pl.kernel,
    mesh=_mesh,
    compiler_params=_sc_params,
    out_type=[
        jax.ShapeDtypeStruct((S, D), jnp.float32),
        jax.ShapeDtypeStruct((S, D), jnp.float32),
    ],
    scratch_types=[
        pltpu.VMEM((CT,), jnp.int32),
        pltpu.VMEM((L,), jnp.int32),
        pltpu.VMEM((CT, D), jnp.float32),
        pltpu.SemaphoreType.DMA,
    ],
)
def _k2_gather_x(x_hbm, idx0_hbm, idx1_hbm, cnt_hbm, x0_hbm, x1_hbm,
                 idxv, cntv, rows, sem):
    wid = _wid()
    base = wid * CT
    iota = _iota()
    pltpu.sync_copy(cnt_hbm.at[pl.ds(wid * L, L)], cntv)
    cv = cntv[...]
    for lane, idx_hbm, dst_hbm in ((0, idx0_hbm, x0_hbm),
                                   (1, idx1_hbm, x1_hbm)):
        cnt = jnp.max(jnp.where(iota == lane, cv, 0))
        pltpu.sync_copy(idx_hbm.at[pl.ds(base, CT)], idxv)
        for k in range(4):
            @pl.when(k * 32 < cnt)
            def _(k=k, dst_hbm=dst_hbm):
                pltpu.async_copy(x_hbm.at[idxv.at[pl.ds(k * 32, 32)]],
                                 rows.at[pl.ds(k * 32, 32)], sem).wait()
                pltpu.sync_copy(rows.at[pl.ds(k * 32, 32)],
                                dst_hbm.at[pl.ds(base + k * 32, 32)])


@functools.partial(
    pl.kernel,
    mesh=_mesh,
    compiler_params=_sc_params,
    out_type=jax.ShapeDtypeStruct((S, OUT), jnp.float32),
    scratch_types=[
        pltpu.VMEM((CT,), jnp.int32),
        pltpu.VMEM((L,), jnp.int32),
        pltpu.VMEM((L,), jnp.int32),
        pltpu.VMEM((CT, OUT), jnp.float32),
        pltpu.SemaphoreType.DMA,
    ],
)
def _k4_gather_f(o0_hbm, fp_hbm, prev_hbm, cnt_hbm, f_hbm, fpv, prevv, cntv,
                 rows, sem):
    wid = _wid()
    base = wid * CT
    iota = _iota()
    pltpu.sync_copy(fp_hbm.at[pl.ds(base, CT)], fpv)
    pltpu.sync_copy(prev_hbm.at[pl.ds(wid * L, L)], prevv)
    pltpu.sync_copy(cnt_hbm.at[pl.ds(wid * L, L)], cntv)
    pv = prevv[...]
    cnt1 = jnp.max(jnp.where(iota == 1, cntv[...], 0))

    def fix(j, _):
        fv = fpv[pl.ds(j * L, L)]
        fpv[pl.ds(j * L, L)] = jnp.where(fv < 0, pv, fv)
        return 0

    lax.fori_loop(0, CT // L, fix, 0)
    for k in range(4):
        @pl.when(k * 32 < cnt1)
        def _(k=k):
            pltpu.async_copy(o0_hbm.at[fpv.at[pl.ds(k * 32, 32)]],
                             rows.at[pl.ds(k * 32, 32)], sem).wait()
            pltpu.sync_copy(rows.at[pl.ds(k * 32, 32)],
                            f_hbm.at[pl.ds(base + k * 32, 32)])


@functools.partial(
    pl.kernel,
    mesh=_mesh,
    compiler_params=_sc_params,
    out_type=jax.ShapeDtypeStruct((T, OUT), jnp.float32),
    scratch_types=[
        pltpu.VMEM((UC, UW), jnp.int32),
        pltpu.VMEM((UC, UW), jnp.int32),
        pltpu.VMEM((UC, UW), jnp.int32),
        pltpu.VMEM((UC, UW), jnp.float32),
        pltpu.VMEM((L,), jnp.int32),
        pltpu.VMEM((UW, OUT), jnp.float32),
        pltpu.VMEM((UW, OUT), jnp.float32),
        pltpu.SemaphoreType.DMA,
        pltpu.SemaphoreType.DMA,
    ],
)
def _k6_combine(o0_hbm, o1_hbm, idxU_hbm, pU0_hbm, pU1_hbm, wU_hbm, cnt_hbm,
                out_hbm, idxUv, p0v, p1v, wv, cntv, arows, brows, gsem, zsem):
    wid = _wid()
    base = wid * CHUNK
    iota = _iota()
    zf = jnp.zeros((L,), jnp.float32)
    pltpu.sync_copy(idxU_hbm.at[wid], idxUv)
    pltpu.sync_copy(pU0_hbm.at[wid], p0v)
    pltpu.sync_copy(pU1_hbm.at[wid], p1v)
    pltpu.sync_copy(wU_hbm.at[wid], wv)
    pltpu.sync_copy(cnt_hbm.at[pl.ds(wid * L, L)], cntv)
    cntU = jnp.max(jnp.where(iota == 2, cntv[...], 0))

    for r in range(UW):
        def zcol(c, _, r=r):
            arows[r, pl.ds(c * L, L)] = zf
            return 0

        lax.fori_loop(0, OUT // L, zcol, 0)

    zcopies = [
        pltpu.async_copy(arows, out_hbm.at[pl.ds(base + k * UW, UW)], zsem)
        for k in range(ZN)
    ]

    for c in range(UC):
        if c == 0:
            gb = pltpu.async_copy(o1_hbm.at[p1v.at[0]], brows, gsem)
            gb.wait()

            def rowfn0(r, _):
                ws = plsc.load_gather(
                    wv, [jnp.zeros((L,), jnp.int32),
                         jnp.zeros((L,), jnp.int32) + r])

                def col(cc, _):
                    b = brows[r, pl.ds(cc * L, L)]
                    brows[r, pl.ds(cc * L, L)] = b - ws * b
                    return 0

                lax.fori_loop(0, OUT // L, col, 0)
                return 0

            lax.fori_loop(0, UW, rowfn0, 0)
            for z in zcopies:
                z.wait()
            ga = pltpu.async_copy(o0_hbm.at[p0v.at[0]], arows, gsem)
            ga.wait()

            def rowfn1(r, _):
                ws = plsc.load_gather(
                    wv, [jnp.zeros((L,), jnp.int32),
                         jnp.zeros((L,), jnp.int32) + r])

                def col(cc, _):
                    a = arows[r, pl.ds(cc * L, L)]
                    b = brows[r, pl.ds(cc * L, L)]
                    arows[r, pl.ds(cc * L, L)] = b + ws * a
                    return 0

                lax.fori_loop(0, OUT // L, col, 0)
                return 0

            lax.fori_loop(0, UW, rowfn1, 0)
            pltpu.async_copy(arows, out_hbm.at[idxUv.at[0]], gsem).wait()
        else:
            @pl.when(c * UW < cntU)
            def _(c=c):
                ga = pltpu.async_copy(o0_hbm.at[p0v.at[c]], arows, gsem)
                gb = pltpu.async_copy(o1_hbm.at[p1v.at[c]], brows, gsem)
                ga.wait()
                gb.wait()

                def rowfn(r, _, c=c):
                    ws = plsc.load_gather(
                        wv, [jnp.zeros((L,), jnp.int32) + c,
                             jnp.zeros((L,), jnp.int32) + r])

                    def col(cc, _):
                        a = arows[r, pl.ds(cc * L, L)]
                        b = brows[r, pl.ds(cc * L, L)]
                        arows[r, pl.ds(cc * L, L)] = b + ws * (a - b)
                        return 0

                    lax.fori_loop(0, OUT // L, col, 0)
                    return 0

                lax.fori_loop(0, UW, rowfn, 0)
                pltpu.async_copy(arows, out_hbm.at[idxUv.at[c]], gsem).wait()


def _mm0_body(x_ref, w_ref, b_ref, o_ref):
    i = pl.program_id(0)

    @pl.when(i == 0)
    def _():
        o_ref[...] = jnp.zeros_like(o_ref)

    @pl.when(i > 0)
    def _():
        o_ref[...] = (
            jnp.dot(x_ref[...], w_ref[...],
                    preferred_element_type=jnp.float32) + b_ref[...])


def _mm0(x0, w0, b0):
    return pl.pallas_call(
        _mm0_body,
        grid=(S // BM + 1,),
        in_specs=[
            pl.BlockSpec((BM, D), lambda i: (jnp.maximum(i - 1, 0), 0)),
            pl.BlockSpec((D, OUT), lambda i: (0, 0)),
            pl.BlockSpec((1, OUT), lambda i: (0, 0)),
        ],
        out_specs=pl.BlockSpec((BM, OUT), lambda i: (i, 0)),
        out_shape=jax.ShapeDtypeStruct((SP, OUT), jnp.float32),
    )(x0, w0, b0)


def _mm1_body(x_ref, f_ref, w_ref, b_ref, o_ref):
    i = pl.program_id(0)

    @pl.when(i == 0)
    def _():
        o_ref[...] = jnp.zeros_like(o_ref)

    @pl.when(i > 0)
    def _():
        w = w_ref[...]
        o_ref[...] = (
            jnp.dot(x_ref[...], w[:D], preferred_element_type=jnp.float32)
            + jnp.dot(f_ref[...], w[D:], preferred_element_type=jnp.float32)
            + b_ref[...])


def _mm1(x1, f, w1, b1):
    return pl.pallas_call(
        _mm1_body,
        grid=(S // BM + 1,),
        in_specs=[
            pl.BlockSpec((BM, D), lambda i: (jnp.maximum(i - 1, 0), 0)),
            pl.BlockSpec((BM, OUT), lambda i: (jnp.maximum(i - 1, 0), 0)),
            pl.BlockSpec((D + OUT, OUT), lambda i: (0, 0)),
            pl.BlockSpec((1, OUT), lambda i: (0, 0)),
        ],
        out_specs=pl.BlockSpec((BM, OUT), lambda i: (i, 0)),
        out_shape=jax.ShapeDtypeStruct((SP, OUT), jnp.float32),
    )(x1, f, w1, b1)


_BLK = 256


def _fused_body(t0_ref, t1_ref, w_ref, x_ref, W0_ref, b0_ref, W1_ref, b1_ref,
                out_ref, carry_ref, *, blk, d):
    i = pl.program_id(0)
    xb = x_ref[...]
    t0 = t0_ref[...]
    t1 = t1_ref[...]
    m0 = (t0 == 0) | (t1 == 0)
    m1 = (t0 == 1) | (t1 == 1)
    o0 = jnp.where(
        m0,
        jnp.dot(xb, W0_ref[...], preferred_element_type=jnp.float32)
        + b0_ref[...],
        0.0,
    )

    @pl.when(i == 0)
    def _():
        carry_ref[...] = o0[0:1, :]

    f = o0
    v = m0.astype(jnp.int32)
    s = 1
    while s < blk:
        f = jnp.where(v > 0, f, jnp.concatenate([f[:s], f[:-s]], axis=0))
        v = jnp.maximum(v, jnp.concatenate([v[:s], v[:-s]], axis=0))
        s *= 2
    filled = jnp.where(v > 0, f, carry_ref[...])
    carry_ref[...] = filled[blk - 1:blk, :]

    o1 = jnp.where(
        m1,
        jnp.dot(xb, W1_ref[0:d, :], preferred_element_type=jnp.float32)
        + jnp.dot(filled, W1_ref[d:, :], preferred_element_type=jnp.float32)
        + b1_ref[...],
        0.0,
    )
    w = w_ref[...]
    out_ref[...] = w * o0 + (1.0 - w) * o1


def _fused_dense(t0, t1, w0, x, W0, b0, W1, b1):
    blk = _BLK
    body = functools.partial(_fused_body, blk=blk, d=D)
    return pl.pallas_call(
        body,
        grid=(T // blk,),
        in_specs=[
            pl.BlockSpec((blk, 1), lambda i: (i, 0)),
            pl.BlockSpec((blk, 1), lambda i: (i, 0)),
            pl.BlockSpec((blk, 1), lambda i: (i, 0)),
            pl.BlockSpec((blk, D), lambda i: (i, 0)),
            pl.BlockSpec((D, OUT), lambda i: (0, 0)),
            pl.BlockSpec((1, OUT), lambda i: (0, 0)),
            pl.BlockSpec((D + OUT, OUT), lambda i: (0, 0)),
            pl.BlockSpec((1, OUT), lambda i: (0, 0)),
        ],
        out_specs=pl.BlockSpec((blk, OUT), lambda i: (i, 0)),
        out_shape=jax.ShapeDtypeStruct((T, OUT), jnp.float32),
        scratch_shapes=[pltpu.VMEM((1, OUT), jnp.float32)],
        compiler_params=pltpu.CompilerParams(
            dimension_semantics=("arbitrary",),
        ),
    )(t0, t1, w0, x, W0, b0, W1, b1)


def kernel(x, topk_idx, weights, W0, b0, W1, b1):
    t0 = topk_idx[:, 0].astype(jnp.int32)
    t1 = topk_idx[:, 1].astype(jnp.int32)
    wcol = weights[:, 0]
    b0r = b0.reshape(1, OUT)
    b1r = b1.reshape(1, OUT)

    (idx0seg, idx1seg, fillpos, idxU, pU0, pU1, wU,
     cnts) = _k1_router(t0, t1, wcol)
    c = cnts.reshape(NW, L)
    cnt0 = c[:, 0]
    cnt1 = c[:, 1]
    cntU = c[:, 2]
    overflow = ((jnp.max(cnt0) > CT) | (jnp.max(cnt1) > CT)
                | (jnp.max(cntU) > CTU))

    pos = jnp.where(cnt0 > 0,
                    BM + jnp.arange(NW, dtype=jnp.int32) * CT
                    + jnp.minimum(cnt0, CT) - 1, 0)
    incl = lax.cummax(pos, axis=0)
    prevpos = jnp.concatenate([jnp.zeros((1,), jnp.int32), incl[:-1]])
    prevvec = jnp.repeat(prevpos, L).astype(jnp.int32)

    def sparse_path():
        x0c, x1c = _k2_gather_x(x, idx0seg, idx1seg, cnts)
        o0pad = _mm0(x0c, W0, b0r)
        f = _k4_gather_f(o0pad, fillpos, prevvec, cnts)
        o1pad = _mm1(x1c, f, W1, b1r)
        return _k6_combine(o0pad, o1pad, idxU, pU0, pU1, wU, cnts)

    def dense_path():
        return _fused_dense(t0.reshape(T, 1), t1.reshape(T, 1),
                            wcol.reshape(T, 1), x, W0, b0r, W1, b1r)

    return lax.cond(overflow, dense_path, sparse_path)


# speedup vs baseline: 1.0632x; 1.0632x over previous
import functools

import jax
import jax.numpy as jnp
from jax import lax
from jax.experimental import pallas as pl
from jax.experimental.pallas import tpu as pltpu
from jax.experimental.pallas import tpu_sc as plsc

T = 32768
D = 768
OUT = 768
NC = 2
NS = 16
NW = NC * NS
CHUNK = T // NW
CT = 128
S = NW * CT
BM = 256
SP = BM + S
L = 16
UC = 4
UW = 64
CTU = UC * UW
ZN = CHUNK // UW

_mesh = plsc.VectorSubcoreMesh(core_axis_name="c", subcore_axis_name="s")
_sc_params = pltpu.CompilerParams(needs_layout_passes=False)


def _wid():
    return lax.axis_index("s") * NC + lax.axis_index("c")


def _iota():
    return lax.iota(jnp.int32, L)


@functools.partial(
    pl.kernel,
    mesh=_mesh,
    compiler_params=_sc_params,
    out_type=[
        jax.ShapeDtypeStruct((S,), jnp.int32),
        jax.ShapeDtypeStruct((S,), jnp.int32),
        jax.ShapeDtypeStruct((S,), jnp.int32),
        jax.ShapeDtypeStruct((NW, UC, UW), jnp.int32),
        jax.ShapeDtypeStruct((NW, UC, UW), jnp.int32),
        jax.ShapeDtypeStruct((NW, UC, UW), jnp.int32),
        jax.ShapeDtypeStruct((NW, UC, UW), jnp.float32),
        jax.ShapeDtypeStruct((NW * L,), jnp.int32),
    ],
    scratch_types=[
        pltpu.VMEM((CHUNK,), jnp.int32),
        pltpu.VMEM((CHUNK,), jnp.int32),
        pltpu.VMEM((CHUNK,), jnp.float32),
        pltpu.VMEM((CT,), jnp.int32),
        pltpu.VMEM((CT,), jnp.int32),
        pltpu.VMEM((CT,), jnp.int32),
        pltpu.VMEM((UC, UW), jnp.int32),
        pltpu.VMEM((UC, UW), jnp.int32),
        pltpu.VMEM((UC, UW), jnp.int32),
        pltpu.VMEM((UC, UW), jnp.float32),
        pltpu.VMEM((L,), jnp.int32),
    ],
)
def _k1_router(t0_hbm, t1_hbm, w_hbm,
               idx0_hbm, idx1_hbm, fp_hbm, idxU_hbm, pU0_hbm, pU1_hbm,
               wU_hbm, cnt_hbm,
               t0v, t1v, wcv, idx0l, idx1l, fpl, idxUl, pU0l, pU1l, wUl,
               infov):
    wid = _wid()
    base = wid * CHUNK
    iota = _iota()
    zi = jnp.zeros((L,), jnp.int32)
    zf = jnp.zeros((L,), jnp.float32)

    pltpu.sync_copy(t0_hbm.at[pl.ds(base, CHUNK)], t0v)
    pltpu.sync_copy(t1_hbm.at[pl.ds(base, CHUNK)], t1v)
    pltpu.sync_copy(w_hbm.at[pl.ds(base, CHUNK)], wcv)

    def init(j, _):
        idx0l[pl.ds(j * L, L)] = zi
        idx1l[pl.ds(j * L, L)] = zi
        fpl[pl.ds(j * L, L)] = zi
        return 0

    lax.fori_loop(0, CT // L, init, 0)

    for cc in range(UC):
        def initu(j, _, cc=cc):
            idxUl[cc, pl.ds(j * L, L)] = base + zi
            pU0l[cc, pl.ds(j * L, L)] = zi
            pU1l[cc, pl.ds(j * L, L)] = zi
            wUl[cc, pl.ds(j * L, L)] = zf
            return 0

        lax.fori_loop(0, UW // L, initu, 0)

    seg_base = BM + wid * CT

    def body(j, carry):
        s0, s1, sU = carry
        off = j * L
        v0 = t0v[pl.ds(off, L)]
        v1 = t1v[pl.ds(off, L)]
        wload = wcv[pl.ds(off, L)]
        m0 = (v0 == 0) | (v1 == 0)
        m1 = (v0 == 1) | (v1 == 1)
        mU = m0 | m1
        c0 = jnp.cumsum(jnp.where(m0, 1, 0)) + s0
        c1 = jnp.cumsum(jnp.where(m1, 1, 0)) + s1
        cU = jnp.cumsum(jnp.where(mU, 1, 0)) + sU
        r0 = c0 - 1
        r1 = c1 - 1
        rU = cU - 1
        tok = base + off + iota
        ok0 = m0 & (r0 < CT)
        ok1 = m1 & (r1 < CT)
        okU = mU & (rU < CTU)
        plsc.store_scatter(idx0l, [r0], tok, mask=ok0)
        plsc.store_scatter(idx1l, [r1], tok, mask=ok1)
        fpv = jnp.where(c0 > 0, BM + wid * CT + jnp.minimum(c0, CT) - 1,
                        -jnp.ones((L,), jnp.int32))
        plsc.store_scatter(fpl, [r1], fpv, mask=ok1)
        hi = lax.shift_right_logical(jnp.maximum(rU, 0), 6)
        lo = jnp.maximum(rU, 0) & 63
        plsc.store_scatter(idxUl, [hi, lo], tok, mask=okU)
        plsc.store_scatter(pU0l, [hi, lo], jnp.where(ok0, seg_base + r0, 0),
                           mask=okU)
        plsc.store_scatter(pU1l, [hi, lo], jnp.where(ok1, seg_base + r1, 0),
                           mask=okU)
        plsc.store_scatter(wUl, [hi, lo], wload, mask=okU)
        s0n = s0 + plsc.all_reduce_population_count(m0)
        s1n = s1 + plsc.all_reduce_population_count(m1)
        sUn = sU + plsc.all_reduce_population_count(mU)
        return s0n, s1n, sUn

    s0, s1, sU = lax.fori_loop(0, CHUNK // L, body, (zi, zi, zi))

    ltv = jnp.maximum(jnp.minimum(sU, CTU) - 1, 0)
    lhi = lax.shift_right_logical(ltv, 6)
    llo = ltv & 63
    lastTok = plsc.load_gather(idxUl, [lhi, llo])
    lastP0 = plsc.load_gather(pU0l, [lhi, llo])
    lastP1 = plsc.load_gather(pU1l, [lhi, llo])
    lastW = plsc.load_gather(wUl, [lhi, llo])
    for cc in range(UC):
        for jj in range(UW // L):
            slot = cc * UW + jj * L + iota
            cond = slot < sU
            sl = pl.ds(jj * L, L)
            idxUl[cc, sl] = jnp.where(cond, idxUl[cc, sl], lastTok)
            pU0l[cc, sl] = jnp.where(cond, pU0l[cc, sl], lastP0)
            pU1l[cc, sl] = jnp.where(cond, pU1l[cc, sl], lastP1)
            wUl[cc, sl] = jnp.where(cond, wUl[cc, sl], lastW)

    infov[...] = jnp.where(
        iota == 0, s0,
        jnp.where(iota == 1, s1, jnp.where(iota == 2, sU, 0)))
    pltpu.sync_copy(idx0l, idx0_hbm.at[pl.ds(wid * CT, CT)])
    pltpu.sync_copy(idx1l, idx1_hbm.at[pl.ds(wid * CT, CT)])
    pltpu.sync_copy(fpl, fp_hbm.at[pl.ds(wid * CT, CT)])
    pltpu.sync_copy(idxUl, idxU_hbm.at[wid])
    pltpu.sync_copy(pU0l, pU0_hbm.at[wid])
    pltpu.sync_copy(pU1l, pU1_hbm.at[wid])
    pltpu.sync_copy(wUl, wU_hbm.at[wid])
    pltpu.sync_copy(infov, cnt_hbm.at[pl.ds(wid * L, L)])


@functools.partial(
    pl.kernel,
    mesh=_mesh,
    compiler_params=_sc_params,
    out_type=[
        jax.ShapeDtypeStruct((S, D), jnp.float32),
        jax.ShapeDtypeStruct((S, D), jnp.float32),
    ],
    scratch_types=[
        pltpu.VMEM((CT,), jnp.int32),
        pltpu.VMEM((L,), jnp.int32),
        pltpu.VMEM((CT, D), jnp.float32),
        pltpu.SemaphoreType.DMA,
    ],
)
def _k2_gather_x(x_hbm, idx0_hbm, idx1_hbm, cnt_hbm, x0_hbm, x1_hbm,
                 idxv, cntv, rows, sem):
    wid = _wid()
    base = wid * CT
    iota = _iota()
    pltpu.sync_copy(cnt_hbm.at[pl.ds(wid * L, L)], cntv)
    cv = cntv[...]
    for lane, idx_hbm, dst_hbm in ((0, idx0_hbm, x0_hbm),
                                   (1, idx1_hbm, x1_hbm)):
        cnt = jnp.max(jnp.where(iota == lane, cv, 0))
        pltpu.sync_copy(idx_hbm.at[pl.ds(base, CT)], idxv)
        for k in range(4):
            @pl.when(k * 32 < cnt)
            def _(k=k, dst_hbm=dst_hbm):
                pltpu.async_copy(x_hbm.at[idxv.at[pl.ds(k * 32, 32)]],
                                 rows.at[pl.ds(k * 32, 32)], sem).wait()
                pltpu.sync_copy(rows.at[pl.ds(k * 32, 32)],
                                dst_hbm.at[pl.ds(base + k * 32, 32)])


@functools.partial(
    pl.kernel,
    mesh=_mesh,
    compiler_params=_sc_params,
    out_type=jax.ShapeDtypeStruct((S, OUT), jnp.float32),
    scratch_types=[
        pltpu.VMEM((CT,), jnp.int32),
        pltpu.VMEM((L,), jnp.int32),
        pltpu.VMEM((L,), jnp.int32),
        pltpu.VMEM((CT, OUT), jnp.float32),
        pltpu.SemaphoreType.DMA,
    ],
)
def _k4_gather_f(o0_hbm, fp_hbm, prev_hbm, cnt_hbm, f_hbm, fpv, prevv, cntv,
                 rows, sem):
    wid = _wid()
    base = wid * CT
    iota = _iota()
    pltpu.sync_copy(fp_hbm.at[pl.ds(base, CT)], fpv)
    pltpu.sync_copy(prev_hbm.at[pl.ds(wid * L, L)], prevv)
    pltpu.sync_copy(cnt_hbm.at[pl.ds(wid * L, L)], cntv)
    pv = prevv[...]
    cnt1 = jnp.max(jnp.where(iota == 1, cntv[...], 0))

    def fix(j, _):
        fv = fpv[pl.ds(j * L, L)]
        fpv[pl.ds(j * L, L)] = jnp.where(fv < 0, pv, fv)
        return 0

    lax.fori_loop(0, CT // L, fix, 0)
    for k in range(4):
        @pl.when(k * 32 < cnt1)
        def _(k=k):
            pltpu.async_copy(o0_hbm.at[fpv.at[pl.ds(k * 32, 32)]],
                             rows.at[pl.ds(k * 32, 32)], sem).wait()
            pltpu.sync_copy(rows.at[pl.ds(k * 32, 32)],
                            f_hbm.at[pl.ds(base + k * 32, 32)])


@functools.partial(
    pl.kernel,
    mesh=_mesh,
    compiler_params=_sc_params,
    out_type=jax.ShapeDtypeStruct((T, OUT), jnp.float32),
    scratch_types=[
        pltpu.VMEM((UC, UW), jnp.int32),
        pltpu.VMEM((UC, UW), jnp.int32),
        pltpu.VMEM((UC, UW), jnp.int32),
        pltpu.VMEM((UC, UW), jnp.float32),
        pltpu.VMEM((L,), jnp.int32),
        pltpu.VMEM((UW, OUT), jnp.float32),
        pltpu.VMEM((UW, OUT), jnp.float32),
        pltpu.VMEM((UW // 2, OUT), jnp.float32),
        pltpu.SemaphoreType.DMA,
        pltpu.SemaphoreType.DMA,
    ],
)
def _k6_combine(o0_hbm, o1_hbm, idxU_hbm, pU0_hbm, pU1_hbm, wU_hbm, cnt_hbm,
                out_hbm, idxUv, p0v, p1v, wv, cntv, arows, brows, zrows,
                gsem, zsem):
    wid = _wid()
    base = wid * CHUNK
    iota = _iota()
    zf = jnp.zeros((L,), jnp.float32)
    pltpu.sync_copy(idxU_hbm.at[wid], idxUv)
    pltpu.sync_copy(pU0_hbm.at[wid], p0v)
    pltpu.sync_copy(pU1_hbm.at[wid], p1v)
    pltpu.sync_copy(wU_hbm.at[wid], wv)
    pltpu.sync_copy(cnt_hbm.at[pl.ds(wid * L, L)], cntv)
    cntU = jnp.max(jnp.where(iota == 2, cntv[...], 0))

    for r in range(UW // 2):
        def zcol(c, _, r=r):
            zrows[r, pl.ds(c * L, L)] = zf
            return 0

        lax.fori_loop(0, OUT // L, zcol, 0)

    zcopies = [
        pltpu.async_copy(zrows, out_hbm.at[pl.ds(base + k * (UW // 2),
                                                 UW // 2)], zsem)
        for k in range(2 * ZN)
    ]

    for c in range(UC):
        @pl.when(c * UW < cntU)
        def _(c=c):
            ga = pltpu.async_copy(o0_hbm.at[p0v.at[c]], arows, gsem)
            gb = pltpu.async_copy(o1_hbm.at[p1v.at[c]], brows, gsem)
            ga.wait()
            gb.wait()

            def rowfn(r, _, c=c):
                ws = plsc.load_gather(
                    wv, [jnp.zeros((L,), jnp.int32) + c,
                         jnp.zeros((L,), jnp.int32) + r])

                def col(cc, _):
                    a = arows[r, pl.ds(cc * L, L)]
                    b = brows[r, pl.ds(cc * L, L)]
                    arows[r, pl.ds(cc * L, L)] = b + ws * (a - b)
                    return 0

                lax.fori_loop(0, OUT // L, col, 0)
                return 0

            lax.fori_loop(0, UW, rowfn, 0)

        if c == 0:
            for z in zcopies:
                z.wait()

        @pl.when(c * UW < cntU)
        def _(c=c):
            pltpu.async_copy(arows, out_hbm.at[idxUv.at[c]], gsem).wait()


def _mm0_body(x_ref, w_ref, b_ref, o_ref):
    i = pl.program_id(0)

    @pl.when(i == 0)
    def _():
        o_ref[...] = jnp.zeros_like(o_ref)

    @pl.when(i > 0)
    def _():
        o_ref[...] = (
            jnp.dot(x_ref[...], w_ref[...],
                    preferred_element_type=jnp.float32) + b_ref[...])


def _mm0(x0, w0, b0):
    return pl.pallas_call(
        _mm0_body,
        grid=(S // BM + 1,),
        in_specs=[
            pl.BlockSpec((BM, D), lambda i: (jnp.maximum(i - 1, 0), 0)),
            pl.BlockSpec((D, OUT), lambda i: (0, 0)),
            pl.BlockSpec((1, OUT), lambda i: (0, 0)),
        ],
        out_specs=pl.BlockSpec((BM, OUT), lambda i: (i, 0)),
        out_shape=jax.ShapeDtypeStruct((SP, OUT), jnp.float32),
    )(x0, w0, b0)


def _mm1_body(x_ref, f_ref, w_ref, b_ref, o_ref):
    i = pl.program_id(0)

    @pl.when(i == 0)
    def _():
        o_ref[...] = jnp.zeros_like(o_ref)

    @pl.when(i > 0)
    def _():
        w = w_ref[...]
        o_ref[...] = (
            jnp.dot(x_ref[...], w[:D], preferred_element_type=jnp.float32)
            + jnp.dot(f_ref[...], w[D:], preferred_element_type=jnp.float32)
            + b_ref[...])


def _mm1(x1, f, w1, b1):
    return pl.pallas_call(
        _mm1_body,
        grid=(S // BM + 1,),
        in_specs=[
            pl.BlockSpec((BM, D), lambda i: (jnp.maximum(i - 1, 0), 0)),
            pl.BlockSpec((BM, OUT), lambda i: (jnp.maximum(i - 1, 0), 0)),
            pl.BlockSpec((D + OUT, OUT), lambda i: (0, 0)),
            pl.BlockSpec((1, OUT), lambda i: (0, 0)),
        ],
        out_specs=pl.BlockSpec((BM, OUT), lambda i: (i, 0)),
        out_shape=jax.ShapeDtypeStruct((SP, OUT), jnp.float32),
    )(x1, f, w1, b1)


_BLK = 256


def _fused_body(t0_ref, t1_ref, w_ref, x_ref, W0_ref, b0_ref, W1_ref, b1_ref,
                out_ref, carry_ref, *, blk, d):
    i = pl.program_id(0)
    xb = x_ref[...]
    t0 = t0_ref[...]
    t1 = t1_ref[...]
    m0 = (t0 == 0) | (t1 == 0)
    m1 = (t0 == 1) | (t1 == 1)
    o0 = jnp.where(
        m0,
        jnp.dot(xb, W0_ref[...], preferred_element_type=jnp.float32)
        + b0_ref[...],
        0.0,
    )

    @pl.when(i == 0)
    def _():
        carry_ref[...] = o0[0:1, :]

    f = o0
    v = m0.astype(jnp.int32)
    s = 1
    while s < blk:
        f = jnp.where(v > 0, f, jnp.concatenate([f[:s], f[:-s]], axis=0))
        v = jnp.maximum(v, jnp.concatenate([v[:s], v[:-s]], axis=0))
        s *= 2
    filled = jnp.where(v > 0, f, carry_ref[...])
    carry_ref[...] = filled[blk - 1:blk, :]

    o1 = jnp.where(
        m1,
        jnp.dot(xb, W1_ref[0:d, :], preferred_element_type=jnp.float32)
        + jnp.dot(filled, W1_ref[d:, :], preferred_element_type=jnp.float32)
        + b1_ref[...],
        0.0,
    )
    w = w_ref[...]
    out_ref[...] = w * o0 + (1.0 - w) * o1


def _fused_dense(t0, t1, w0, x, W0, b0, W1, b1):
    blk = _BLK
    body = functools.partial(_fused_body, blk=blk, d=D)
    return pl.pallas_call(
        body,
        grid=(T // blk,),
        in_specs=[
            pl.BlockSpec((blk, 1), lambda i: (i, 0)),
            pl.BlockSpec((blk, 1), lambda i: (i, 0)),
            pl.BlockSpec((blk, 1), lambda i: (i, 0)),
            pl.BlockSpec((blk, D), lambda i: (i, 0)),
            pl.BlockSpec((D, OUT), lambda i: (0, 0)),
            pl.BlockSpec((1, OUT), lambda i: (0, 0)),
            pl.BlockSpec((D + OUT, OUT), lambda i: (0, 0)),
            pl.BlockSpec((1, OUT), lambda i: (0, 0)),
        ],
        out_specs=pl.BlockSpec((blk, OUT), lambda i: (i, 0)),
        out_shape=jax.ShapeDtypeStruct((T, OUT), jnp.float32),
        scratch_shapes=[pltpu.VMEM((1, OUT), jnp.float32)],
        compiler_params=pltpu.CompilerParams(
            dimension_semantics=("arbitrary",),
        ),
    )(t0, t1, w0, x, W0, b0, W1, b1)


def kernel(x, topk_idx, weights, W0, b0, W1, b1):
    t0 = topk_idx[:, 0].astype(jnp.int32)
    t1 = topk_idx[:, 1].astype(jnp.int32)
    wcol = weights[:, 0]
    b0r = b0.reshape(1, OUT)
    b1r = b1.reshape(1, OUT)

    (idx0seg, idx1seg, fillpos, idxU, pU0, pU1, wU,
     cnts) = _k1_router(t0, t1, wcol)
    c = cnts.reshape(NW, L)
    cnt0 = c[:, 0]
    cnt1 = c[:, 1]
    cntU = c[:, 2]
    overflow = ((jnp.max(cnt0) > CT) | (jnp.max(cnt1) > CT)
                | (jnp.max(cntU) > CTU))

    pos = jnp.where(cnt0 > 0,
                    BM + jnp.arange(NW, dtype=jnp.int32) * CT
                    + jnp.minimum(cnt0, CT) - 1, 0)
    incl = lax.cummax(pos, axis=0)
    prevpos = jnp.concatenate([jnp.zeros((1,), jnp.int32), incl[:-1]])
    prevvec = jnp.repeat(prevpos, L).astype(jnp.int32)

    def sparse_path():
        x0c, x1c = _k2_gather_x(x, idx0seg, idx1seg, cnts)
        o0pad = _mm0(x0c, W0, b0r)
        f = _k4_gather_f(o0pad, fillpos, prevvec, cnts)
        o1pad = _mm1(x1c, f, W1, b1r)
        return _k6_combine(o0pad, o1pad, idxU, pU0, pU1, wU, cnts)

    def dense_path():
        return _fused_dense(t0.reshape(T, 1), t1.reshape(T, 1),
                            wcol.reshape(T, 1), x, W0, b0r, W1, b1r)

    return lax.cond(overflow, dense_path, sparse_path)


# speedup vs baseline: 1.1503x; 1.0820x over previous
import functools

import jax
import jax.numpy as jnp
from jax import lax
from jax.experimental import pallas as pl
from jax.experimental.pallas import tpu as pltpu
from jax.experimental.pallas import tpu_sc as plsc

T = 32768
D = 768
OUT = 768
NC = 2
NS = 16
NW = NC * NS
CHUNK = T // NW
CT = 64
S = NW * CT
BM = 256
SP = BM + S
L = 16
UC = 2
UW = 64
CTU = UC * UW
ZN = CHUNK // UW

_mesh = plsc.VectorSubcoreMesh(core_axis_name="c", subcore_axis_name="s")
_sc_params = pltpu.CompilerParams(needs_layout_passes=False)


def _wid():
    return lax.axis_index("s") * NC + lax.axis_index("c")


def _iota():
    return lax.iota(jnp.int32, L)


@functools.partial(
    pl.kernel,
    mesh=_mesh,
    compiler_params=_sc_params,
    out_type=[
        jax.ShapeDtypeStruct((S,), jnp.int32),
        jax.ShapeDtypeStruct((S,), jnp.int32),
        jax.ShapeDtypeStruct((S,), jnp.int32),
        jax.ShapeDtypeStruct((NW, UC, UW), jnp.int32),
        jax.ShapeDtypeStruct((NW, UC, UW), jnp.int32),
        jax.ShapeDtypeStruct((NW, UC, UW), jnp.int32),
        jax.ShapeDtypeStruct((NW, UC, UW), jnp.float32),
        jax.ShapeDtypeStruct((NW * L,), jnp.int32),
    ],
    scratch_types=[
        pltpu.VMEM((CHUNK,), jnp.int32),
        pltpu.VMEM((CHUNK,), jnp.int32),
        pltpu.VMEM((CHUNK,), jnp.float32),
        pltpu.VMEM((CT,), jnp.int32),
        pltpu.VMEM((CT,), jnp.int32),
        pltpu.VMEM((CT,), jnp.int32),
        pltpu.VMEM((UC, UW), jnp.int32),
        pltpu.VMEM((UC, UW), jnp.int32),
        pltpu.VMEM((UC, UW), jnp.int32),
        pltpu.VMEM((UC, UW), jnp.float32),
        pltpu.VMEM((L,), jnp.int32),
    ],
)
def _k1_router(t0_hbm, t1_hbm, w_hbm,
               idx0_hbm, idx1_hbm, fp_hbm, idxU_hbm, pU0_hbm, pU1_hbm,
               wU_hbm, cnt_hbm,
               t0v, t1v, wcv, idx0l, idx1l, fpl, idxUl, pU0l, pU1l, wUl,
               infov):
    wid = _wid()
    base = wid * CHUNK
    iota = _iota()
    zi = jnp.zeros((L,), jnp.int32)
    zf = jnp.zeros((L,), jnp.float32)

    pltpu.sync_copy(t0_hbm.at[pl.ds(base, CHUNK)], t0v)
    pltpu.sync_copy(t1_hbm.at[pl.ds(base, CHUNK)], t1v)
    pltpu.sync_copy(w_hbm.at[pl.ds(base, CHUNK)], wcv)

    def init(j, _):
        idx0l[pl.ds(j * L, L)] = zi
        idx1l[pl.ds(j * L, L)] = zi
        fpl[pl.ds(j * L, L)] = zi
        return 0

    lax.fori_loop(0, CT // L, init, 0)

    for cc in range(UC):
        def initu(j, _, cc=cc):
            idxUl[cc, pl.ds(j * L, L)] = base + zi
            pU0l[cc, pl.ds(j * L, L)] = zi
            pU1l[cc, pl.ds(j * L, L)] = zi
            wUl[cc, pl.ds(j * L, L)] = zf
            return 0

        lax.fori_loop(0, UW // L, initu, 0)

    seg_base = BM + wid * CT

    def body(j, carry):
        s0, s1, sU = carry
        off = j * L
        v0 = t0v[pl.ds(off, L)]
        v1 = t1v[pl.ds(off, L)]
        wload = wcv[pl.ds(off, L)]
        m0 = (v0 == 0) | (v1 == 0)
        m1 = (v0 == 1) | (v1 == 1)
        mU = m0 | m1
        c0 = jnp.cumsum(jnp.where(m0, 1, 0)) + s0
        c1 = jnp.cumsum(jnp.where(m1, 1, 0)) + s1
        cU = jnp.cumsum(jnp.where(mU, 1, 0)) + sU
        r0 = c0 - 1
        r1 = c1 - 1
        rU = cU - 1
        tok = base + off + iota
        ok0 = m0 & (r0 < CT)
        ok1 = m1 & (r1 < CT)
        okU = mU & (rU < CTU)
        plsc.store_scatter(idx0l, [r0], tok, mask=ok0)
        plsc.store_scatter(idx1l, [r1], tok, mask=ok1)
        fpv = jnp.where(c0 > 0, BM + wid * CT + jnp.minimum(c0, CT) - 1,
                        -jnp.ones((L,), jnp.int32))
        plsc.store_scatter(fpl, [r1], fpv, mask=ok1)
        hi = lax.shift_right_logical(jnp.maximum(rU, 0), 6)
        lo = jnp.maximum(rU, 0) & 63
        plsc.store_scatter(idxUl, [hi, lo], tok, mask=okU)
        plsc.store_scatter(pU0l, [hi, lo], jnp.where(ok0, seg_base + r0, 0),
                           mask=okU)
        plsc.store_scatter(pU1l, [hi, lo], jnp.where(ok1, seg_base + r1, 0),
                           mask=okU)
        plsc.store_scatter(wUl, [hi, lo], wload, mask=okU)
        s0n = s0 + plsc.all_reduce_population_count(m0)
        s1n = s1 + plsc.all_reduce_population_count(m1)
        sUn = sU + plsc.all_reduce_population_count(mU)
        return s0n, s1n, sUn

    s0, s1, sU = lax.fori_loop(0, CHUNK // L, body, (zi, zi, zi))

    ltv = jnp.maximum(jnp.minimum(sU, CTU) - 1, 0)
    lhi = lax.shift_right_logical(ltv, 6)
    llo = ltv & 63
    lastTok = plsc.load_gather(idxUl, [lhi, llo])
    lastP0 = plsc.load_gather(pU0l, [lhi, llo])
    lastP1 = plsc.load_gather(pU1l, [lhi, llo])
    lastW = plsc.load_gather(wUl, [lhi, llo])
    for cc in range(UC):
        for jj in range(UW // L):
            slot = cc * UW + jj * L + iota
            cond = slot < sU
            sl = pl.ds(jj * L, L)
            idxUl[cc, sl] = jnp.where(cond, idxUl[cc, sl], lastTok)
            pU0l[cc, sl] = jnp.where(cond, pU0l[cc, sl], lastP0)
            pU1l[cc, sl] = jnp.where(cond, pU1l[cc, sl], lastP1)
            wUl[cc, sl] = jnp.where(cond, wUl[cc, sl], lastW)

    infov[...] = jnp.where(
        iota == 0, s0,
        jnp.where(iota == 1, s1, jnp.where(iota == 2, sU, 0)))
    pltpu.sync_copy(idx0l, idx0_hbm.at[pl.ds(wid * CT, CT)])
    pltpu.sync_copy(idx1l, idx1_hbm.at[pl.ds(wid * CT, CT)])
    pltpu.sync_copy(fpl, fp_hbm.at[pl.ds(wid * CT, CT)])
    pltpu.sync_copy(idxUl, idxU_hbm.at[wid])
    pltpu.sync_copy(pU0l, pU0_hbm.at[wid])
    pltpu.sync_copy(pU1l, pU1_hbm.at[wid])
    pltpu.sync_copy(wUl, wU_hbm.at[wid])
    pltpu.sync_copy(infov, cnt_hbm.at[pl.ds(wid * L, L)])


@functools.partial(
    pl.kernel,
    mesh=_mesh,
    compiler_params=_sc_params,
    out_type=[
        jax.ShapeDtypeStruct((S, D), jnp.float32),
        jax.ShapeDtypeStruct((S, D), jnp.float32),
    ],
    scratch_types=[
        pltpu.VMEM((CT,), jnp.int32),
        pltpu.VMEM((L,), jnp.int32),
        pltpu.VMEM((CT, D), jnp.float32),
        pltpu.SemaphoreType.DMA,
    ],
)
def _k2_gather_x(x_hbm, idx0_hbm, idx1_hbm, cnt_hbm, x0_hbm, x1_hbm,
                 idxv, cntv, rows, sem):
    wid = _wid()
    base = wid * CT
    iota = _iota()
    pltpu.sync_copy(cnt_hbm.at[pl.ds(wid * L, L)], cntv)
    cv = cntv[...]
    for lane, idx_hbm, dst_hbm in ((0, idx0_hbm, x0_hbm),
                                   (1, idx1_hbm, x1_hbm)):
        cnt = jnp.max(jnp.where(iota == lane, cv, 0))
        pltpu.sync_copy(idx_hbm.at[pl.ds(base, CT)], idxv)
        for k in range(CT // 32):
            @pl.when(k * 32 < cnt)
            def _(k=k, dst_hbm=dst_hbm):
                pltpu.async_copy(x_hbm.at[idxv.at[pl.ds(k * 32, 32)]],
                                 rows.at[pl.ds(k * 32, 32)], sem).wait()
                pltpu.sync_copy(rows.at[pl.ds(k * 32, 32)],
                                dst_hbm.at[pl.ds(base + k * 32, 32)])


@functools.partial(
    pl.kernel,
    mesh=_mesh,
    compiler_params=_sc_params,
    out_type=jax.ShapeDtypeStruct((S, OUT), jnp.float32),
    scratch_types=[
        pltpu.VMEM((CT,), jnp.int32),
        pltpu.VMEM((L,), jnp.int32),
        pltpu.VMEM((L,), jnp.int32),
        pltpu.VMEM((CT, OUT), jnp.float32),
        pltpu.SemaphoreType.DMA,
    ],
)
def _k4_gather_f(o0_hbm, fp_hbm, prev_hbm, cnt_hbm, f_hbm, fpv, prevv, cntv,
                 rows, sem):
    wid = _wid()
    base = wid * CT
    iota = _iota()
    pltpu.sync_copy(fp_hbm.at[pl.ds(base, CT)], fpv)
    pltpu.sync_copy(prev_hbm.at[pl.ds(wid * L, L)], prevv)
    pltpu.sync_copy(cnt_hbm.at[pl.ds(wid * L, L)], cntv)
    pv = prevv[...]
    cnt1 = jnp.max(jnp.where(iota == 1, cntv[...], 0))

    def fix(j, _):
        fv = fpv[pl.ds(j * L, L)]
        fpv[pl.ds(j * L, L)] = jnp.where(fv < 0, pv, fv)
        return 0

    lax.fori_loop(0, CT // L, fix, 0)
    for k in range(CT // 32):
        @pl.when(k * 32 < cnt1)
        def _(k=k):
            pltpu.async_copy(o0_hbm.at[fpv.at[pl.ds(k * 32, 32)]],
                             rows.at[pl.ds(k * 32, 32)], sem).wait()
            pltpu.sync_copy(rows.at[pl.ds(k * 32, 32)],
                            f_hbm.at[pl.ds(base + k * 32, 32)])


@functools.partial(
    pl.kernel,
    mesh=_mesh,
    compiler_params=_sc_params,
    out_type=jax.ShapeDtypeStruct((T, OUT), jnp.float32),
    scratch_types=[
        pltpu.VMEM((UC, UW), jnp.int32),
        pltpu.VMEM((UC, UW), jnp.int32),
        pltpu.VMEM((UC, UW), jnp.int32),
        pltpu.VMEM((UC, UW), jnp.float32),
        pltpu.VMEM((L,), jnp.int32),
        pltpu.VMEM((UW, OUT), jnp.float32),
        pltpu.VMEM((UW, OUT), jnp.float32),
        pltpu.VMEM((UW // 2, OUT), jnp.float32),
        pltpu.SemaphoreType.DMA,
        pltpu.SemaphoreType.DMA,
    ],
)
def _k6_combine(o0_hbm, o1_hbm, idxU_hbm, pU0_hbm, pU1_hbm, wU_hbm, cnt_hbm,
                out_hbm, idxUv, p0v, p1v, wv, cntv, arows, brows, zrows,
                gsem, zsem):
    wid = _wid()
    base = wid * CHUNK
    iota = _iota()
    zf = jnp.zeros((L,), jnp.float32)
    pltpu.sync_copy(idxU_hbm.at[wid], idxUv)
    pltpu.sync_copy(pU0_hbm.at[wid], p0v)
    pltpu.sync_copy(pU1_hbm.at[wid], p1v)
    pltpu.sync_copy(wU_hbm.at[wid], wv)
    pltpu.sync_copy(cnt_hbm.at[pl.ds(wid * L, L)], cntv)
    cntU = jnp.max(jnp.where(iota == 2, cntv[...], 0))

    for r in range(UW // 2):
        def zcol(c, _, r=r):
            zrows[r, pl.ds(c * L, L)] = zf
            return 0

        lax.fori_loop(0, OUT // L, zcol, 0)

    zcopies = [
        pltpu.async_copy(zrows, out_hbm.at[pl.ds(base + k * (UW // 2),
                                                 UW // 2)], zsem)
        for k in range(2 * ZN)
    ]

    for c in range(UC):
        @pl.when(c * UW < cntU)
        def _(c=c):
            ga = pltpu.async_copy(o0_hbm.at[p0v.at[c]], arows, gsem)
            gb = pltpu.async_copy(o1_hbm.at[p1v.at[c]], brows, gsem)
            ga.wait()
            gb.wait()

            def rowfn(r, _, c=c):
                ws = plsc.load_gather(
                    wv, [jnp.zeros((L,), jnp.int32) + c,
                         jnp.zeros((L,), jnp.int32) + r])

                def col(cc, _):
                    a = arows[r, pl.ds(cc * L, L)]
                    b = brows[r, pl.ds(cc * L, L)]
                    arows[r, pl.ds(cc * L, L)] = b + ws * (a - b)
                    return 0

                lax.fori_loop(0, OUT // L, col, 0)
                return 0

            lax.fori_loop(0, UW, rowfn, 0)

        if c == 0:
            for z in zcopies:
                z.wait()

        @pl.when(c * UW < cntU)
        def _(c=c):
            pltpu.async_copy(arows, out_hbm.at[idxUv.at[c]], gsem).wait()


def _mm0_body(x_ref, w_ref, b_ref, o_ref):
    i = pl.program_id(0)

    @pl.when(i == 0)
    def _():
        o_ref[...] = jnp.zeros_like(o_ref)

    @pl.when(i > 0)
    def _():
        o_ref[...] = (
            jnp.dot(x_ref[...], w_ref[...],
                    preferred_element_type=jnp.float32) + b_ref[...])


def _mm0(x0, w0, b0):
    return pl.pallas_call(
        _mm0_body,
        grid=(S // BM + 1,),
        in_specs=[
            pl.BlockSpec((BM, D), lambda i: (jnp.maximum(i - 1, 0), 0)),
            pl.BlockSpec((D, OUT), lambda i: (0, 0)),
            pl.BlockSpec((1, OUT), lambda i: (0, 0)),
        ],
        out_specs=pl.BlockSpec((BM, OUT), lambda i: (i, 0)),
        out_shape=jax.ShapeDtypeStruct((SP, OUT), jnp.float32),
    )(x0, w0, b0)


def _mm1_body(x_ref, f_ref, w_ref, b_ref, o_ref):
    i = pl.program_id(0)

    @pl.when(i == 0)
    def _():
        o_ref[...] = jnp.zeros_like(o_ref)

    @pl.when(i > 0)
    def _():
        w = w_ref[...]
        o_ref[...] = (
            jnp.dot(x_ref[...], w[:D], preferred_element_type=jnp.float32)
            + jnp.dot(f_ref[...], w[D:], preferred_element_type=jnp.float32)
            + b_ref[...])


def _mm1(x1, f, w1, b1):
    return pl.pallas_call(
        _mm1_body,
        grid=(S // BM + 1,),
        in_specs=[
            pl.BlockSpec((BM, D), lambda i: (jnp.maximum(i - 1, 0), 0)),
            pl.BlockSpec((BM, OUT), lambda i: (jnp.maximum(i - 1, 0), 0)),
            pl.BlockSpec((D + OUT, OUT), lambda i: (0, 0)),
            pl.BlockSpec((1, OUT), lambda i: (0, 0)),
        ],
        out_specs=pl.BlockSpec((BM, OUT), lambda i: (i, 0)),
        out_shape=jax.ShapeDtypeStruct((SP, OUT), jnp.float32),
    )(x1, f, w1, b1)


_BLK = 256


def _fused_body(t0_ref, t1_ref, w_ref, x_ref, W0_ref, b0_ref, W1_ref, b1_ref,
                out_ref, carry_ref, *, blk, d):
    i = pl.program_id(0)
    xb = x_ref[...]
    t0 = t0_ref[...]
    t1 = t1_ref[...]
    m0 = (t0 == 0) | (t1 == 0)
    m1 = (t0 == 1) | (t1 == 1)
    o0 = jnp.where(
        m0,
        jnp.dot(xb, W0_ref[...], preferred_element_type=jnp.float32)
        + b0_ref[...],
        0.0,
    )

    @pl.when(i == 0)
    def _():
        carry_ref[...] = o0[0:1, :]

    f = o0
    v = m0.astype(jnp.int32)
    s = 1
    while s < blk:
        f = jnp.where(v > 0, f, jnp.concatenate([f[:s], f[:-s]], axis=0))
        v = jnp.maximum(v, jnp.concatenate([v[:s], v[:-s]], axis=0))
        s *= 2
    filled = jnp.where(v > 0, f, carry_ref[...])
    carry_ref[...] = filled[blk - 1:blk, :]

    o1 = jnp.where(
        m1,
        jnp.dot(xb, W1_ref[0:d, :], preferred_element_type=jnp.float32)
        + jnp.dot(filled, W1_ref[d:, :], preferred_element_type=jnp.float32)
        + b1_ref[...],
        0.0,
    )
    w = w_ref[...]
    out_ref[...] = w * o0 + (1.0 - w) * o1


def _fused_dense(t0, t1, w0, x, W0, b0, W1, b1):
    blk = _BLK
    body = functools.partial(_fused_body, blk=blk, d=D)
    return pl.pallas_call(
        body,
        grid=(T // blk,),
        in_specs=[
            pl.BlockSpec((blk, 1), lambda i: (i, 0)),
            pl.BlockSpec((blk, 1), lambda i: (i, 0)),
            pl.BlockSpec((blk, 1), lambda i: (i, 0)),
            pl.BlockSpec((blk, D), lambda i: (i, 0)),
            pl.BlockSpec((D, OUT), lambda i: (0, 0)),
            pl.BlockSpec((1, OUT), lambda i: (0, 0)),
            pl.BlockSpec((D + OUT, OUT), lambda i: (0, 0)),
            pl.BlockSpec((1, OUT), lambda i: (0, 0)),
        ],
        out_specs=pl.BlockSpec((blk, OUT), lambda i: (i, 0)),
        out_shape=jax.ShapeDtypeStruct((T, OUT), jnp.float32),
        scratch_shapes=[pltpu.VMEM((1, OUT), jnp.float32)],
        compiler_params=pltpu.CompilerParams(
            dimension_semantics=("arbitrary",),
        ),
    )(t0, t1, w0, x, W0, b0, W1, b1)


def kernel(x, topk_idx, weights, W0, b0, W1, b1):
    t0 = topk_idx[:, 0].astype(jnp.int32)
    t1 = topk_idx[:, 1].astype(jnp.int32)
    wcol = weights[:, 0]
    b0r = b0.reshape(1, OUT)
    b1r = b1.reshape(1, OUT)

    (idx0seg, idx1seg, fillpos, idxU, pU0, pU1, wU,
     cnts) = _k1_router(t0, t1, wcol)
    c = cnts.reshape(NW, L)
    cnt0 = c[:, 0]
    cnt1 = c[:, 1]
    cntU = c[:, 2]
    overflow = ((jnp.max(cnt0) > CT) | (jnp.max(cnt1) > CT)
                | (jnp.max(cntU) > CTU))

    pos = jnp.where(cnt0 > 0,
                    BM + jnp.arange(NW, dtype=jnp.int32) * CT
                    + jnp.minimum(cnt0, CT) - 1, 0)
    incl = lax.cummax(pos, axis=0)
    prevpos = jnp.concatenate([jnp.zeros((1,), jnp.int32), incl[:-1]])
    prevvec = jnp.repeat(prevpos, L).astype(jnp.int32)

    def sparse_path():
        x0c, x1c = _k2_gather_x(x, idx0seg, idx1seg, cnts)
        o0pad = _mm0(x0c, W0, b0r)
        f = _k4_gather_f(o0pad, fillpos, prevvec, cnts)
        o1pad = _mm1(x1c, f, W1, b1r)
        return _k6_combine(o0pad, o1pad, idxU, pU0, pU1, wU, cnts)

    def dense_path():
        return _fused_dense(t0.reshape(T, 1), t1.reshape(T, 1),
                            wcol.reshape(T, 1), x, W0, b0r, W1, b1r)

    return lax.cond(overflow, dense_path, sparse_path)
